# Initial kernel scaffold; baseline (speedup 1.0000x reference)
import jax
import jax.numpy as jnp
from jax.experimental import pallas as pl


def kernel(x, edge_index, edge_attr, params):
    raise NotImplementedError("write your pallas kernel here")


# speedup vs baseline: 1.0341x; 1.0341x over previous
import functools

import jax
import jax.numpy as jnp
from jax.experimental import pallas as pl
from jax.experimental.pallas import tpu as pltpu

N = 10000
E = 320000
D = 128
DE = 16
PAD = 144

NB = 2000
EB = 8000


def _relu(v):
    return jnp.maximum(v, 0.0)


def _dot(a, b):
    return jnp.dot(a, b, preferred_element_type=jnp.float32)


def _proj_body(x_ref, w_ref, b_ref, o_ref):
    o_ref[...] = _relu(_dot(x_ref[...], w_ref[...]) + b_ref[...])


def _precomp_body(h_ref, wmh_ref, wqs_ref, wqd_ref, ms_ref, qs_ref, qd_ref):
    h = h_ref[...]
    ms_ref[...] = _dot(h, wmh_ref[...])
    qs_ref[...] = _dot(h, wqs_ref[...])
    qd_ref[...] = _dot(h, wqd_ref[...])


def _update_body(h_ref, agg_ref, w1h_ref, w1a_ref, b1_ref, w2_ref, b2_ref,
                 g_ref, lb_ref, o_ref):
    h = h_ref[...]
    aggp = agg_ref[...]
    tot = aggp[0] + aggp[1]
    num = tot[:, :D]
    den = tot[:, D:D + 1]
    agg = num / (den + 1e-6)
    u = _relu(_dot(h, w1h_ref[...]) + _dot(agg, w1a_ref[...]) + b1_ref[...])
    out = _dot(u, w2_ref[...]) + b2_ref[...]
    z = _relu(out + h)
    mu = jnp.mean(z, axis=-1, keepdims=True)
    var = jnp.mean((z - mu) ** 2, axis=-1, keepdims=True)
    o_ref[...] = (z - mu) / jnp.sqrt(var + 1e-5) * g_ref[...] + lb_ref[...]


def _qhead_body(h_ref, w1_ref, b1_ref, w2_ref, b2_ref, o_ref):
    u = _relu(_dot(h_ref[...], w1_ref[...]) + b1_ref[...])
    o_ref[...] = _dot(u, w2_ref[...]) + b2_ref[...]


def _score_body(qs_ref, qd_ref, ea_ref, wae_ref, ab1_ref, wa2_ref, ab2_ref,
                s_ref, smax_ref):
    a = qs_ref[...] + qd_ref[...] + _dot(ea_ref[...], wae_ref[...]) + ab1_ref[...]
    a = jnp.where(a > 0, a, 0.2 * a)
    s = _dot(a, wa2_ref[...]) + ab2_ref[...]
    s_ref[...] = s

    @pl.when(pl.program_id(0) == 0)
    def _():
        smax_ref[...] = jnp.full((1, 1), -1e30, jnp.float32)

    smax_ref[...] = jnp.maximum(smax_ref[...], jnp.max(s))


def _msg_body(ms_ref, ea_ref, wme_ref, mb1_ref, wm2_ref, mb2_ref, s_ref,
              smax_ref, o_ref):
    m = _relu(ms_ref[...] + _dot(ea_ref[...], wme_ref[...]) + mb1_ref[...])
    msg = _dot(m, wm2_ref[...]) + mb2_ref[...]
    e = jnp.exp(s_ref[...] - smax_ref[...])
    o_ref[...] = jnp.concatenate(
        [msg * e, e, jnp.zeros((msg.shape[0], PAD - D - 1), jnp.float32)], axis=1)


def _full(shape):
    return pl.BlockSpec(shape, lambda i: tuple(0 for _ in shape))


def _proj(x, w, b):
    return pl.pallas_call(
        _proj_body,
        grid=(N // NB,),
        in_specs=[pl.BlockSpec((NB, D), lambda i: (i, 0)), _full((D, D)),
                  _full((1, D))],
        out_specs=pl.BlockSpec((NB, D), lambda i: (i, 0)),
        out_shape=jax.ShapeDtypeStruct((N, D), jnp.float32),
    )(x, w, b)


def _precomp(h, wmh, wqs, wqd):
    return pl.pallas_call(
        _precomp_body,
        grid=(N // NB,),
        in_specs=[pl.BlockSpec((NB, D), lambda i: (i, 0)), _full((D, D)),
                  _full((D, D // 2)), _full((D, D // 2))],
        out_specs=[pl.BlockSpec((NB, D), lambda i: (i, 0)),
                   pl.BlockSpec((NB, D // 2), lambda i: (i, 0)),
                   pl.BlockSpec((NB, D // 2), lambda i: (i, 0))],
        out_shape=[jax.ShapeDtypeStruct((N, D), jnp.float32),
                   jax.ShapeDtypeStruct((N, D // 2), jnp.float32),
                   jax.ShapeDtypeStruct((N, D // 2), jnp.float32)],
    )(h, wmh, wqs, wqd)


def _scores(qs_e, qd_e, ea, wae, ab1, wa2, ab2):
    return pl.pallas_call(
        _score_body,
        grid=(E // EB,),
        in_specs=[
            pl.BlockSpec((EB, D // 2), lambda i: (i, 0)),
            pl.BlockSpec((EB, D // 2), lambda i: (i, 0)),
            pl.BlockSpec((EB, DE), lambda i: (i, 0)),
            _full((DE, D // 2)), _full((1, D // 2)),
            _full((D // 2, 1)), _full((1, 1)),
        ],
        out_specs=[pl.BlockSpec((EB, 1), lambda i: (i, 0)),
                   pl.BlockSpec((1, 1), lambda i: (0, 0))],
        out_shape=[jax.ShapeDtypeStruct((E, 1), jnp.float32),
                   jax.ShapeDtypeStruct((1, 1), jnp.float32)],
    )(qs_e, qd_e, ea, wae, ab1, wa2, ab2)


def _messages(ms_e, ea, wme, mb1, wm2, mb2, s, smax):
    return pl.pallas_call(
        _msg_body,
        grid=(E // EB,),
        in_specs=[
            pl.BlockSpec((EB, D), lambda i: (i, 0)),
            pl.BlockSpec((EB, DE), lambda i: (i, 0)),
            _full((DE, D)), _full((1, D)),
            _full((D, D)), _full((1, D)),
            pl.BlockSpec((EB, 1), lambda i: (i, 0)),
            pl.BlockSpec((1, 1), lambda i: (0, 0)),
        ],
        out_specs=pl.BlockSpec((EB, PAD), lambda i: (i, 0)),
        out_shape=jax.ShapeDtypeStruct((E, PAD), jnp.float32),
    )(ms_e, ea, wme, mb1, wm2, mb2, s, smax)


def _update(h, aggp, w1h, w1a, b1, w2, b2, g, lb):
    return pl.pallas_call(
        _update_body,
        grid=(N // NB,),
        in_specs=[
            pl.BlockSpec((NB, D), lambda i: (i, 0)),
            pl.BlockSpec((2, NB, PAD), lambda i: (0, i, 0)),
            _full((D, D)), _full((D, D)), _full((1, D)),
            _full((D, D)), _full((1, D)), _full((1, D)), _full((1, D)),
        ],
        out_specs=pl.BlockSpec((NB, D), lambda i: (i, 0)),
        out_shape=jax.ShapeDtypeStruct((N, D), jnp.float32),
    )(h, aggp, w1h, w1a, b1, w2, b2, g, lb)


def _qhead(h, w1, b1, w2, b2):
    return pl.pallas_call(
        _qhead_body,
        grid=(N // NB,),
        in_specs=[pl.BlockSpec((NB, D), lambda i: (i, 0)), _full((D, D)),
                  _full((1, D)), _full((D, 1)), _full((1, 1))],
        out_specs=pl.BlockSpec((NB, 1), lambda i: (i, 0)),
        out_shape=jax.ShapeDtypeStruct((N, 1), jnp.float32),
    )(h, w1, b1, w2, b2)


def _gather(ms, qs, qd, src, dst):
    ms_e = jnp.take(ms, src, axis=0)
    qs_e = jnp.take(qs, src, axis=0)
    qd_e = jnp.take(qd, dst, axis=0)
    return ms_e, qs_e, qd_e


def _scatter(wmsg_pad, dst):
    seg = jax.ops.segment_sum(wmsg_pad, dst, num_segments=N)
    return jnp.stack([seg, jnp.zeros_like(seg)])


def kernel(x, edge_index, edge_attr, params):
    src = edge_index[0]
    dst = edge_index[1]
    h = _proj(x, params['proj_w'].T, params['proj_b'][None, :])
    for p in params['layers']:
        mw1 = p['msg_w1']
        wmh = mw1[:, :D].T
        wme = mw1[:, D:].T
        aw1 = p['attn_w1']
        wqs = aw1[:, :D].T
        wqd = aw1[:, D:2 * D].T
        wae = aw1[:, 2 * D:].T
        uw1 = p['upd_w1']
        w1h = uw1[:, :D].T
        w1a = uw1[:, D:].T

        ms, qs, qd = _precomp(h, wmh, wqs, wqd)
        ms_e, qs_e, qd_e = _gather(ms, qs, qd, src, dst)
        s, smax = _scores(qs_e, qd_e, edge_attr, wae,
                          p['attn_b1'][None, :], p['attn_w2'].T,
                          p['attn_b2'][None, :])
        wmsg_pad = _messages(ms_e, edge_attr, wme, p['msg_b1'][None, :],
                             p['msg_w2'].T, p['msg_b2'][None, :], s, smax)
        aggp = _scatter(wmsg_pad, dst)
        h = _update(h, aggp, w1h, w1a, p['upd_b1'][None, :],
                    p['upd_w2'].T, p['upd_b2'][None, :],
                    p['ln_g'][None, :], p['ln_b'][None, :])
    q = _qhead(h, params['q_w1'].T, params['q_b1'][None, :],
               params['q_w2'].T, params['q_b2'][None, :])
    return q[:, 0]


# speedup vs baseline: 3.1235x; 3.0206x over previous
import functools

import jax
import jax.numpy as jnp
from jax import lax
from jax.experimental import pallas as pl
from jax.experimental.pallas import tpu as pltpu
from jax.experimental.pallas import tpu_sc as plsc

N = 10000
E = 320000
D = 128
DE = 16
DH = D // 2

NB = 2000
EB = 8000

NC = 2
NS = 16
NW = NC * NS
EPW = E // NW
GC = 80
NCH = EPW // GC
NP = 10240
NPT = NP // NS


def _relu(v):
    return jnp.maximum(v, 0.0)


def _dot(a, b):
    return jnp.dot(a, b, preferred_element_type=jnp.float32)


def _proj_body(x_ref, w_ref, b_ref, o_ref):
    o_ref[...] = _relu(_dot(x_ref[...], w_ref[...]) + b_ref[...])


def _precomp_body(h_ref, wmh_ref, wqs_ref, wqd_ref, ms_ref, qq_ref):
    h = h_ref[...]
    ms_ref[...] = _dot(h, wmh_ref[...])
    qq_ref[...] = jnp.concatenate(
        [_dot(h, wqs_ref[...]), _dot(h, wqd_ref[...])], axis=1)


def _denred_body(d_ref, o_ref):
    @pl.when(pl.program_id(1) == 0)
    def _():
        o_ref[...] = jnp.zeros_like(o_ref)

    o_ref[...] += d_ref[...][0]


def _update_body(h_ref, agg_ref, den_ref, w1h_ref, w1a_ref, b1_ref, w2_ref,
                 b2_ref, g_ref, lb_ref, o_ref):
    h = h_ref[...]
    aggp = agg_ref[...]
    num = aggp[0] + aggp[1]
    den = den_ref[...]
    agg = num / (den + 1e-6)
    u = _relu(_dot(h, w1h_ref[...]) + _dot(agg, w1a_ref[...]) + b1_ref[...])
    out = _dot(u, w2_ref[...]) + b2_ref[...]
    z = _relu(out + h)
    mu = jnp.mean(z, axis=-1, keepdims=True)
    var = jnp.mean((z - mu) ** 2, axis=-1, keepdims=True)
    o_ref[...] = (z - mu) / jnp.sqrt(var + 1e-5) * g_ref[...] + lb_ref[...]


def _qhead_body(h_ref, w1_ref, b1_ref, w2_ref, b2_ref, o_ref):
    u = _relu(_dot(h_ref[...], w1_ref[...]) + b1_ref[...])
    o_ref[...] = _dot(u, w2_ref[...]) + b2_ref[...]


def _edge_body(ms_ref, qqs_ref, qqd_ref, ea_ref, wae_ref, ab1_ref, wa2_ref,
               ab2_ref, wme_ref, mb1_ref, wm2_ref, mb2_ref, w_ref, e_ref):
    qs = qqs_ref[...][:, :DH]
    qd = qqd_ref[...][:, DH:]
    ea = ea_ref[...]
    a = qs + qd + _dot(ea, wae_ref[...]) + ab1_ref[...]
    a = jnp.where(a > 0, a, 0.2 * a)
    s = _dot(a, wa2_ref[...]) + ab2_ref[...]
    e = jnp.exp(s)
    m = _relu(ms_ref[...] + _dot(ea, wme_ref[...]) + mb1_ref[...])
    msg = _dot(m, wm2_ref[...]) + mb2_ref[...]
    w_ref[...] = msg * e
    e_ref[...] = e


def _full(shape):
    return pl.BlockSpec(shape, lambda i: tuple(0 for _ in shape))


def _proj(x, w, b):
    return pl.pallas_call(
        _proj_body,
        grid=(N // NB,),
        in_specs=[pl.BlockSpec((NB, D), lambda i: (i, 0)), _full((D, D)),
                  _full((1, D))],
        out_specs=pl.BlockSpec((NB, D), lambda i: (i, 0)),
        out_shape=jax.ShapeDtypeStruct((N, D), jnp.float32),
    )(x, w, b)


def _precomp(h, wmh, wqs, wqd):
    return pl.pallas_call(
        _precomp_body,
        grid=(N // NB,),
        in_specs=[pl.BlockSpec((NB, D), lambda i: (i, 0)), _full((D, D)),
                  _full((D, DH)), _full((D, DH))],
        out_specs=[pl.BlockSpec((NB, D), lambda i: (i, 0)),
                   pl.BlockSpec((NB, D), lambda i: (i, 0))],
        out_shape=[jax.ShapeDtypeStruct((N, D), jnp.float32),
                   jax.ShapeDtypeStruct((N, D), jnp.float32)],
    )(h, wmh, wqs, wqd)


def _edges(ms_e, qqs_e, qqd_e, ea, wae, ab1, wa2, ab2, wme, mb1, wm2, mb2):
    return pl.pallas_call(
        _edge_body,
        grid=(E // EB,),
        in_specs=[
            pl.BlockSpec((EB, D), lambda i: (i, 0)),
            pl.BlockSpec((EB, D), lambda i: (i, 0)),
            pl.BlockSpec((EB, D), lambda i: (i, 0)),
            pl.BlockSpec((EB, DE), lambda i: (i, 0)),
            _full((DE, DH)), _full((1, DH)),
            _full((DH, 1)), _full((1, 1)),
            _full((DE, D)), _full((1, D)),
            _full((D, D)), _full((1, D)),
        ],
        out_specs=[pl.BlockSpec((EB, D), lambda i: (i, 0)),
                   pl.BlockSpec((EB, 1), lambda i: (i, 0))],
        out_shape=[jax.ShapeDtypeStruct((E, D), jnp.float32),
                   jax.ShapeDtypeStruct((E, 1), jnp.float32)],
    )(ms_e, qqs_e, qqd_e, ea, wae, ab1, wa2, ab2, wme, mb1, wm2, mb2)


def _denred(den5):
    return pl.pallas_call(
        _denred_body,
        grid=(N // NB, NW),
        in_specs=[pl.BlockSpec((1, 1, NB, 1), lambda i, w: (w, i, 0, 0))],
        out_specs=pl.BlockSpec((1, NB, 1), lambda i, w: (i, 0, 0)),
        out_shape=jax.ShapeDtypeStruct((N // NB, NB, 1), jnp.float32),
    )(den5)


def _update(h, aggp, den, w1h, w1a, b1, w2, b2, g, lb):
    return pl.pallas_call(
        _update_body,
        grid=(N // NB,),
        in_specs=[
            pl.BlockSpec((NB, D), lambda i: (i, 0)),
            pl.BlockSpec((NC, NB, D), lambda i: (0, i, 0)),
            pl.BlockSpec((NB, 1), lambda i: (i, 0)),
            _full((D, D)), _full((D, D)), _full((1, D)),
            _full((D, D)), _full((1, D)), _full((1, D)), _full((1, D)),
        ],
        out_specs=pl.BlockSpec((NB, D), lambda i: (i, 0)),
        out_shape=jax.ShapeDtypeStruct((N, D), jnp.float32),
    )(h, aggp, den, w1h, w1a, b1, w2, b2, g, lb)


def _qhead(h, w1, b1, w2, b2):
    return pl.pallas_call(
        _qhead_body,
        grid=(N // NB,),
        in_specs=[pl.BlockSpec((NB, D), lambda i: (i, 0)), _full((D, D)),
                  _full((1, D)), _full((D, 1)), _full((1, 1))],
        out_specs=pl.BlockSpec((NB, 1), lambda i: (i, 0)),
        out_shape=jax.ShapeDtypeStruct((N, 1), jnp.float32),
    )(h, w1, b1, w2, b2)


def _sc_mesh():
    return plsc.VectorSubcoreMesh(core_axis_name="c", subcore_axis_name="s",
                                  num_cores=NC, num_subcores=NS)


def _sc_gather(ms, qq, src3, dst3):
    @functools.partial(
        pl.kernel,
        out_type=[jax.ShapeDtypeStruct((E, D), jnp.float32),
                  jax.ShapeDtypeStruct((E, D), jnp.float32),
                  jax.ShapeDtypeStruct((E, D), jnp.float32)],
        mesh=_sc_mesh(),
        scratch_types=[
            pltpu.VMEM((NCH, GC), jnp.int32),
            pltpu.VMEM((NCH, GC), jnp.int32),
            pltpu.VMEM((GC, D), jnp.float32),
            pltpu.VMEM((GC, D), jnp.float32),
            pltpu.VMEM((GC, D), jnp.float32),
            pltpu.SemaphoreType.DMA,
            pltpu.SemaphoreType.DMA,
            pltpu.SemaphoreType.DMA,
        ],
    )
    def k(ms_hbm, qq_hbm, src_hbm, dst_hbm, mse_hbm, qqse_hbm, qqde_hbm,
          sidx, didx, buf1, buf2, buf3, sem1, sem2, sem3):
        wid = lax.axis_index("s") * NC + lax.axis_index("c")
        base = wid * EPW
        pltpu.sync_copy(src_hbm.at[wid], sidx)
        pltpu.sync_copy(dst_hbm.at[wid], didx)

        def body(j, carry):
            cp1 = pltpu.async_copy(ms_hbm.at[sidx.at[j]], buf1, sem1)
            cp2 = pltpu.async_copy(qq_hbm.at[sidx.at[j]], buf2, sem2)
            cp3 = pltpu.async_copy(qq_hbm.at[didx.at[j]], buf3, sem3)
            cp1.wait()
            cp2.wait()
            cp3.wait()
            off = base + j * GC
            pltpu.sync_copy(buf1, mse_hbm.at[pl.ds(off, GC)])
            pltpu.sync_copy(buf2, qqse_hbm.at[pl.ds(off, GC)])
            pltpu.sync_copy(buf3, qqde_hbm.at[pl.ds(off, GC)])
            return carry

        lax.fori_loop(0, NCH, body, 0)

    return k(ms, qq, src3, dst3)


def _sc_scatter(wmsg, e3, dst3):
    @functools.partial(
        pl.kernel,
        out_type=[jax.ShapeDtypeStruct((NC, NP, D), jnp.float32),
                  jax.ShapeDtypeStruct((NC, NS, NP), jnp.float32)],
        mesh=_sc_mesh(),
        scratch_types=[
            pltpu.VMEM((NCH, GC), jnp.int32),
            pltpu.VMEM((GC,), jnp.float32),
            pltpu.VMEM((GC, D), jnp.float32),
            pltpu.VMEM((NP,), jnp.float32),
            pltpu.VMEM_SHARED((NP, D), jnp.float32),
            pltpu.SemaphoreType.DMA,
        ],
        compiler_params=pltpu.CompilerParams(needs_layout_passes=False),
    )
    def k(w_hbm, e_hbm, dst_hbm, out_hbm, den_hbm,
          didx, ebuf, wbuf, den, acc, sem):
        cid = lax.axis_index("c")
        sid = lax.axis_index("s")
        wid = sid * NC + cid
        base = wid * EPW
        zv = jnp.zeros((16,), jnp.float32)

        def zbody(i, carry):
            for c in range(D // 16):
                wbuf[i, pl.ds(c * 16, 16)] = zv
            return carry

        lax.fori_loop(0, GC, zbody, 0)
        for r in range(NPT // GC):
            pltpu.sync_copy(wbuf, acc.at[pl.ds(sid * NPT + r * GC, GC)])

        def zbody2(i, carry):
            den[pl.ds(i * 16, 16)] = zv
            return carry

        lax.fori_loop(0, NP // 16, zbody2, 0)

        pltpu.sync_copy(dst_hbm.at[wid], didx)
        plsc.subcore_barrier()

        def body(j, carry):
            cp = pltpu.async_copy(
                w_hbm.at[pl.ds(base + j * GC, GC)], wbuf, sem)
            pltpu.sync_copy(e_hbm.at[wid, j], ebuf)
            for g in range(GC // 16):
                idxv = didx[j, pl.ds(g * 16, 16)]
                ev = ebuf[pl.ds(g * 16, 16)]
                plsc.addupdate_scatter(den, [idxv], ev)
            cp.wait()
            pltpu.sync_copy(wbuf, acc.at[didx.at[j]], add=True)
            return carry

        lax.fori_loop(0, NCH, body, 0)
        plsc.subcore_barrier()
        pltpu.sync_copy(acc.at[pl.ds(sid * NPT, NPT)],
                        out_hbm.at[cid, pl.ds(sid * NPT, NPT)])
        pltpu.sync_copy(den, den_hbm.at[cid, sid])

    return k(wmsg, e3, dst3)


def kernel(x, edge_index, edge_attr, params):
    src3 = edge_index[0].reshape(NW, NCH, GC)
    dst3 = edge_index[1].reshape(NW, NCH, GC)
    h = _proj(x, params['proj_w'].T, params['proj_b'][None, :])
    for p in params['layers']:
        mw1 = p['msg_w1']
        wmh = mw1[:, :D].T
        wme = mw1[:, D:].T
        aw1 = p['attn_w1']
        wqs = aw1[:, :D].T
        wqd = aw1[:, D:2 * D].T
        wae = aw1[:, 2 * D:].T
        uw1 = p['upd_w1']
        w1h = uw1[:, :D].T
        w1a = uw1[:, D:].T

        ms, qq = _precomp(h, wmh, wqs, wqd)
        ms_e, qqs_e, qqd_e = _sc_gather(ms, qq, src3, dst3)
        wmsg, e = _edges(ms_e, qqs_e, qqd_e, edge_attr, wae,
                         p['attn_b1'][None, :], p['attn_w2'].T,
                         p['attn_b2'][None, :], wme, p['msg_b1'][None, :],
                         p['msg_w2'].T, p['msg_b2'][None, :])
        e3 = e.reshape(NW, NCH, GC)
        aggp, denp = _sc_scatter(wmsg, e3, dst3)
        den5 = denp.reshape(NW, NP)[:, :N].reshape(NW, N // NB, NB, 1)
        den = _denred(den5).reshape(N, 1)
        h = _update(h, aggp, den, w1h, w1a, p['upd_b1'][None, :],
                    p['upd_w2'].T, p['upd_b2'][None, :],
                    p['ln_g'][None, :], p['ln_b'][None, :])
    q = _qhead(h, params['q_w1'].T, params['q_b1'][None, :],
               params['q_w2'].T, params['q_b2'][None, :])
    return q[:, 0]


# speedup vs baseline: 3.2756x; 1.0487x over previous
import functools

import jax
import jax.numpy as jnp
from jax import lax
from jax.experimental import pallas as pl
from jax.experimental.pallas import tpu as pltpu
from jax.experimental.pallas import tpu_sc as plsc

N = 10000
E = 320000
D = 128
DE = 16
DH = D // 2

NB = 2000
EB = 8000

NC = 2
NS = 16
NW = NC * NS
EPW = E // NW
GC = 80
NCH = EPW // GC
NP = 10240
NPT = NP // NS


def _relu(v):
    return jnp.maximum(v, 0.0)


def _dot(a, b):
    return jnp.dot(a, b, preferred_element_type=jnp.float32)


def _proj_body(x_ref, w_ref, b_ref, o_ref):
    o_ref[...] = _relu(_dot(x_ref[...], w_ref[...]) + b_ref[...])


def _precomp_body(h_ref, wmh_ref, wqs_ref, wqd_ref, ms_ref, qq_ref):
    h = h_ref[...]
    ms_ref[...] = _dot(h, wmh_ref[...])
    qq_ref[...] = jnp.concatenate(
        [_dot(h, wqs_ref[...]), _dot(h, wqd_ref[...])], axis=1)


def _denred_body(d_ref, o_ref):
    @pl.when(pl.program_id(1) == 0)
    def _():
        o_ref[...] = jnp.zeros_like(o_ref)

    o_ref[...] += d_ref[...][0]


def _update_body(h_ref, agg_ref, den_ref, w1h_ref, w1a_ref, b1_ref, w2_ref,
                 b2_ref, g_ref, lb_ref, o_ref):
    h = h_ref[...]
    aggp = agg_ref[...]
    num = aggp[0] + aggp[1]
    den = den_ref[...]
    agg = num / (den + 1e-6)
    u = _relu(_dot(h, w1h_ref[...]) + _dot(agg, w1a_ref[...]) + b1_ref[...])
    out = _dot(u, w2_ref[...]) + b2_ref[...]
    z = _relu(out + h)
    mu = jnp.mean(z, axis=-1, keepdims=True)
    var = jnp.mean((z - mu) ** 2, axis=-1, keepdims=True)
    o_ref[...] = (z - mu) / jnp.sqrt(var + 1e-5) * g_ref[...] + lb_ref[...]


def _qhead_body(h_ref, w1_ref, b1_ref, w2_ref, b2_ref, o_ref):
    u = _relu(_dot(h_ref[...], w1_ref[...]) + b1_ref[...])
    o_ref[...] = _dot(u, w2_ref[...]) + b2_ref[...]


def _edge_body(ms_ref, qqs_ref, qqd_ref, ea_ref, wae_ref, ab1_ref, wa2_ref,
               ab2_ref, wme_ref, mb1_ref, wm2_ref, mb2_ref, w_ref, e_ref):
    qs = qqs_ref[...][:, :DH]
    qd = qqd_ref[...][:, DH:]
    ea = ea_ref[...]
    a = qs + qd + _dot(ea, wae_ref[...]) + ab1_ref[...]
    a = jnp.where(a > 0, a, 0.2 * a)
    s = _dot(a, wa2_ref[...]) + ab2_ref[...]
    e = jnp.exp(s)
    m = _relu(ms_ref[...] + _dot(ea, wme_ref[...]) + mb1_ref[...])
    msg = _dot(m, wm2_ref[...]) + mb2_ref[...]
    w_ref[...] = msg * e
    e_ref[...] = e


def _full(shape):
    return pl.BlockSpec(shape, lambda i: tuple(0 for _ in shape))


def _proj(x, w, b):
    return pl.pallas_call(
        _proj_body,
        grid=(N // NB,),
        in_specs=[pl.BlockSpec((NB, D), lambda i: (i, 0)), _full((D, D)),
                  _full((1, D))],
        out_specs=pl.BlockSpec((NB, D), lambda i: (i, 0)),
        out_shape=jax.ShapeDtypeStruct((N, D), jnp.float32),
    )(x, w, b)


def _precomp(h, wmh, wqs, wqd):
    return pl.pallas_call(
        _precomp_body,
        grid=(N // NB,),
        in_specs=[pl.BlockSpec((NB, D), lambda i: (i, 0)), _full((D, D)),
                  _full((D, DH)), _full((D, DH))],
        out_specs=[pl.BlockSpec((NB, D), lambda i: (i, 0)),
                   pl.BlockSpec((NB, D), lambda i: (i, 0))],
        out_shape=[jax.ShapeDtypeStruct((N, D), jnp.float32),
                   jax.ShapeDtypeStruct((N, D), jnp.float32)],
    )(h, wmh, wqs, wqd)


def _edges(ms_e, qqs_e, qqd_e, ea, wae, ab1, wa2, ab2, wme, mb1, wm2, mb2):
    return pl.pallas_call(
        _edge_body,
        grid=(E // EB,),
        in_specs=[
            pl.BlockSpec((EB, D), lambda i: (i, 0)),
            pl.BlockSpec((EB, D), lambda i: (i, 0)),
            pl.BlockSpec((EB, D), lambda i: (i, 0)),
            pl.BlockSpec((EB, DE), lambda i: (i, 0)),
            _full((DE, DH)), _full((1, DH)),
            _full((DH, 1)), _full((1, 1)),
            _full((DE, D)), _full((1, D)),
            _full((D, D)), _full((1, D)),
        ],
        out_specs=[pl.BlockSpec((EB, D), lambda i: (i, 0)),
                   pl.BlockSpec((EB, 1), lambda i: (i, 0))],
        out_shape=[jax.ShapeDtypeStruct((E, D), jnp.float32),
                   jax.ShapeDtypeStruct((E, 1), jnp.float32)],
    )(ms_e, qqs_e, qqd_e, ea, wae, ab1, wa2, ab2, wme, mb1, wm2, mb2)


def _denred(den5):
    return pl.pallas_call(
        _denred_body,
        grid=(N // NB, NW),
        in_specs=[pl.BlockSpec((1, 1, NB, 1), lambda i, w: (w, i, 0, 0))],
        out_specs=pl.BlockSpec((1, NB, 1), lambda i, w: (i, 0, 0)),
        out_shape=jax.ShapeDtypeStruct((N // NB, NB, 1), jnp.float32),
    )(den5)


def _update(h, aggp, den, w1h, w1a, b1, w2, b2, g, lb):
    return pl.pallas_call(
        _update_body,
        grid=(N // NB,),
        in_specs=[
            pl.BlockSpec((NB, D), lambda i: (i, 0)),
            pl.BlockSpec((NC, NB, D), lambda i: (0, i, 0)),
            pl.BlockSpec((NB, 1), lambda i: (i, 0)),
            _full((D, D)), _full((D, D)), _full((1, D)),
            _full((D, D)), _full((1, D)), _full((1, D)), _full((1, D)),
        ],
        out_specs=pl.BlockSpec((NB, D), lambda i: (i, 0)),
        out_shape=jax.ShapeDtypeStruct((N, D), jnp.float32),
    )(h, aggp, den, w1h, w1a, b1, w2, b2, g, lb)


def _qhead(h, w1, b1, w2, b2):
    return pl.pallas_call(
        _qhead_body,
        grid=(N // NB,),
        in_specs=[pl.BlockSpec((NB, D), lambda i: (i, 0)), _full((D, D)),
                  _full((1, D)), _full((D, 1)), _full((1, 1))],
        out_specs=pl.BlockSpec((NB, 1), lambda i: (i, 0)),
        out_shape=jax.ShapeDtypeStruct((N, 1), jnp.float32),
    )(h, w1, b1, w2, b2)


def _sc_mesh():
    return plsc.VectorSubcoreMesh(core_axis_name="c", subcore_axis_name="s",
                                  num_cores=NC, num_subcores=NS)


def _sc_gather(ms, qq, src3, dst3):
    @functools.partial(
        pl.kernel,
        out_type=[jax.ShapeDtypeStruct((E, D), jnp.float32),
                  jax.ShapeDtypeStruct((E, D), jnp.float32),
                  jax.ShapeDtypeStruct((E, D), jnp.float32)],
        mesh=_sc_mesh(),
        scratch_types=[
            pltpu.VMEM((NCH, GC), jnp.int32),
            pltpu.VMEM((NCH, GC), jnp.int32),
            pltpu.VMEM((2, GC, D), jnp.float32),
            pltpu.VMEM((2, GC, D), jnp.float32),
            pltpu.VMEM((2, GC, D), jnp.float32),
            pltpu.SemaphoreType.DMA,
            pltpu.SemaphoreType.DMA,
        ],
    )
    def k(ms_hbm, qq_hbm, src_hbm, dst_hbm, mse_hbm, qqse_hbm, qqde_hbm,
          sidx, didx, buf1, buf2, buf3, sem0, sem1):
        wid = lax.axis_index("s") * NC + lax.axis_index("c")
        base = wid * EPW
        pltpu.sync_copy(src_hbm.at[wid], sidx)
        pltpu.sync_copy(dst_hbm.at[wid], didx)
        sems = (sem0, sem1)

        def fire(j, ph):
            pltpu.async_copy(ms_hbm.at[sidx.at[j]], buf1.at[ph], sems[ph])
            pltpu.async_copy(qq_hbm.at[sidx.at[j]], buf2.at[ph], sems[ph])
            pltpu.async_copy(qq_hbm.at[didx.at[j]], buf3.at[ph], sems[ph])

        def drain(j, ph):
            pltpu.make_async_copy(ms_hbm.at[sidx.at[j]], buf1.at[ph],
                                  sems[ph]).wait()
            pltpu.make_async_copy(qq_hbm.at[sidx.at[j]], buf2.at[ph],
                                  sems[ph]).wait()
            pltpu.make_async_copy(qq_hbm.at[didx.at[j]], buf3.at[ph],
                                  sems[ph]).wait()
            off = base + j * GC
            pltpu.sync_copy(buf1.at[ph], mse_hbm.at[pl.ds(off, GC)])
            pltpu.sync_copy(buf2.at[ph], qqse_hbm.at[pl.ds(off, GC)])
            pltpu.sync_copy(buf3.at[ph], qqde_hbm.at[pl.ds(off, GC)])

        fire(0, 0)
        fire(1, 1)

        def body(jj, carry):
            for ph in range(2):
                j = 2 * jj + ph

                drain(j, ph)

                @pl.when(j + 2 < NCH)
                def _():
                    fire(j + 2, ph)

            return carry

        lax.fori_loop(0, NCH // 2, body, 0)
        drain(NCH - 1, 0)

    return k(ms, qq, src3, dst3)


def _sc_scatter(wmsg, e3, dst3):
    @functools.partial(
        pl.kernel,
        out_type=[jax.ShapeDtypeStruct((NC, NP, D), jnp.float32),
                  jax.ShapeDtypeStruct((NC, NS, NP), jnp.float32)],
        mesh=_sc_mesh(),
        scratch_types=[
            pltpu.VMEM((NCH, GC), jnp.int32),
            pltpu.VMEM((GC,), jnp.float32),
            pltpu.VMEM((GC, D), jnp.float32),
            pltpu.VMEM((NP,), jnp.float32),
            pltpu.VMEM_SHARED((NP, D), jnp.float32),
            pltpu.SemaphoreType.DMA,
        ],
        compiler_params=pltpu.CompilerParams(needs_layout_passes=False),
    )
    def k(w_hbm, e_hbm, dst_hbm, out_hbm, den_hbm,
          didx, ebuf, wbuf, den, acc, sem):
        cid = lax.axis_index("c")
        sid = lax.axis_index("s")
        wid = sid * NC + cid
        base = wid * EPW
        zv = jnp.zeros((16,), jnp.float32)

        def zbody(i, carry):
            for c in range(D // 16):
                wbuf[i, pl.ds(c * 16, 16)] = zv
            return carry

        lax.fori_loop(0, GC, zbody, 0)
        for r in range(NPT // GC):
            pltpu.sync_copy(wbuf, acc.at[pl.ds(sid * NPT + r * GC, GC)])

        def zbody2(i, carry):
            den[pl.ds(i * 16, 16)] = zv
            return carry

        lax.fori_loop(0, NP // 16, zbody2, 0)

        pltpu.sync_copy(dst_hbm.at[wid], didx)
        plsc.subcore_barrier()

        def body(j, carry):
            cp = pltpu.async_copy(
                w_hbm.at[pl.ds(base + j * GC, GC)], wbuf, sem)
            pltpu.sync_copy(e_hbm.at[wid, j], ebuf)
            for g in range(GC // 16):
                idxv = didx[j, pl.ds(g * 16, 16)]
                ev = ebuf[pl.ds(g * 16, 16)]
                plsc.addupdate_scatter(den, [idxv], ev)
            cp.wait()
            pltpu.sync_copy(wbuf, acc.at[didx.at[j]], add=True)
            return carry

        lax.fori_loop(0, NCH, body, 0)
        plsc.subcore_barrier()
        pltpu.sync_copy(acc.at[pl.ds(sid * NPT, NPT)],
                        out_hbm.at[cid, pl.ds(sid * NPT, NPT)])
        pltpu.sync_copy(den, den_hbm.at[cid, sid])

    return k(wmsg, e3, dst3)


def kernel(x, edge_index, edge_attr, params):
    src3 = edge_index[0].reshape(NW, NCH, GC)
    dst3 = edge_index[1].reshape(NW, NCH, GC)
    h = _proj(x, params['proj_w'].T, params['proj_b'][None, :])
    for p in params['layers']:
        mw1 = p['msg_w1']
        wmh = mw1[:, :D].T
        wme = mw1[:, D:].T
        aw1 = p['attn_w1']
        wqs = aw1[:, :D].T
        wqd = aw1[:, D:2 * D].T
        wae = aw1[:, 2 * D:].T
        uw1 = p['upd_w1']
        w1h = uw1[:, :D].T
        w1a = uw1[:, D:].T

        ms, qq = _precomp(h, wmh, wqs, wqd)
        ms_e, qqs_e, qqd_e = _sc_gather(ms, qq, src3, dst3)
        wmsg, e = _edges(ms_e, qqs_e, qqd_e, edge_attr, wae,
                         p['attn_b1'][None, :], p['attn_w2'].T,
                         p['attn_b2'][None, :], wme, p['msg_b1'][None, :],
                         p['msg_w2'].T, p['msg_b2'][None, :])
        e3 = e.reshape(NW, NCH, GC)
        aggp, denp = _sc_scatter(wmsg, e3, dst3)
        den5 = denp.reshape(NW, NP)[:, :N].reshape(NW, N // NB, NB, 1)
        den = _denred(den5).reshape(N, 1)
        h = _update(h, aggp, den, w1h, w1a, p['upd_b1'][None, :],
                    p['upd_w2'].T, p['upd_b2'][None, :],
                    p['ln_g'][None, :], p['ln_b'][None, :])
    q = _qhead(h, params['q_w1'].T, params['q_b1'][None, :],
               params['q_w2'].T, params['q_b2'][None, :])
    return q[:, 0]
